# baseline (speedup 1.0000x reference)
import jax
import jax.numpy as jnp
from jax import lax
from jax.experimental import pallas as pl
from jax.experimental.pallas import tpu as pltpu
from jax.experimental.pallas import tpu_sc as plsc

NC = 2
NS = 16
LANES = 16
WAGG = 128
EDGE_B = 320


def _sc_mesh():
    return plsc.VectorSubcoreMesh(
        core_axis_name="c", subcore_axis_name="s",
        num_cores=NC, num_subcores=NS)


DEGW = 128


def _sc_degree(col, ew, zeros16):
    E = col.shape[0]
    N = zeros16.shape[0]
    per = E // (NC * NS)
    CHUNK = 200
    n_chunks = N // CHUNK
    rounds = (n_chunks + NS - 1) // NS

    @pl.kernel(out_type=jax.ShapeDtypeStruct((NC, N, DEGW), jnp.float32),
               mesh=_sc_mesh(),
               scratch_types=[pltpu.VMEM_SHARED((N, DEGW), jnp.float32),
                              pltpu.VMEM((EDGE_B,), jnp.int32),
                              pltpu.VMEM((EDGE_B,), jnp.float32),
                              pltpu.VMEM((EDGE_B, DEGW), jnp.float32)])
    def deg_kernel(col_hbm, ew_hbm, z_hbm, out_hbm, acc, idc, ewv, rows):
        c = lax.axis_index("c")
        s = lax.axis_index("s")

        for r in range(rounds):
            ci = s + r * NS
            off = pl.multiple_of(ci * CHUNK, 8)

            @pl.when(ci < n_chunks)
            def _():
                pltpu.sync_copy(z_hbm.at[pl.ds(off, CHUNK)],
                                acc.at[pl.ds(off, CHUNK)])

        @pl.loop(0, EDGE_B)
        def _(b):
            for t in range(DEGW // LANES):
                rows[b, pl.ds(t * LANES, LANES)] = jnp.zeros((LANES,),
                                                             jnp.float32)

        plsc.subcore_barrier()

        base0 = (c * NS + s) * per

        @pl.loop(0, per, step=EDGE_B)
        def _(k):
            pltpu.sync_copy(col_hbm.at[pl.ds(base0 + k, EDGE_B)], idc)
            pltpu.sync_copy(ew_hbm.at[pl.ds(base0 + k, EDGE_B)], ewv)

            @pl.loop(0, EDGE_B, step=LANES)
            def _(b0):
                wv = ewv[pl.ds(b0, LANES)]
                for l in range(LANES):
                    rows[b0 + l, pl.ds(0, LANES)] = jnp.full(
                        (LANES,), wv[l], jnp.float32)

            pltpu.sync_copy(rows, acc.at[idc], add=True)

        plsc.subcore_barrier()
        for r in range(rounds):
            ci = s + r * NS
            off = pl.multiple_of(ci * CHUNK, 8)

            @pl.when(ci < n_chunks)
            def _():
                pltpu.sync_copy(acc.at[pl.ds(off, CHUNK)],
                                out_hbm.at[c, pl.ds(off, CHUNK)])

    return deg_kernel(col, ew, zeros16)


def _sc_aggregate(hflat, row_flat, col, ew, zeros, feat_split):
    E = col.shape[0]
    N = zeros.shape[0]
    per = E // NS if feat_split else E // (NC * NS)
    nb = per // EDGE_B
    CHUNK = 200
    n_chunks = N // CHUNK
    rounds = (n_chunks + NS - 1) // NS

    @pl.kernel(out_type=jax.ShapeDtypeStruct((NC, N, WAGG), jnp.float32),
               mesh=_sc_mesh(),
               scratch_types=[pltpu.VMEM_SHARED((N, WAGG), jnp.float32),
                              pltpu.VMEM((EDGE_B, WAGG), jnp.float32),
                              pltpu.VMEM((EDGE_B,), jnp.int32),
                              pltpu.VMEM((EDGE_B,), jnp.int32),
                              pltpu.VMEM((EDGE_B,), jnp.float32)])
    def agg_kernel(h_hbm, row_hbm, col_hbm, ew_hbm, z_hbm, out_hbm,
                   acc, rows, idr, idc, ewv):
        c = lax.axis_index("c")
        s = lax.axis_index("s")

        if feat_split:
            base0 = s * per
            roff = c * E + base0
        else:
            base0 = (c * NS + s) * per
            roff = base0

        for r in range(rounds):
            ci = s + r * NS
            off = pl.multiple_of(ci * CHUNK, 8)

            @pl.when(ci < n_chunks)
            def _():
                pltpu.sync_copy(z_hbm.at[pl.ds(off, CHUNK)],
                                acc.at[pl.ds(off, CHUNK)])
        plsc.subcore_barrier()

        @pl.loop(0, per, step=EDGE_B)
        def _(k):
            pltpu.sync_copy(row_hbm.at[pl.ds(roff + k, EDGE_B)], idr)
            pltpu.sync_copy(col_hbm.at[pl.ds(base0 + k, EDGE_B)], idc)
            pltpu.sync_copy(ew_hbm.at[pl.ds(base0 + k, EDGE_B)], ewv)
            pltpu.sync_copy(h_hbm.at[idr], rows)

            @pl.loop(0, EDGE_B, step=LANES)
            def _(b0):
                wv = ewv[pl.ds(b0, LANES)]
                for l in range(LANES):
                    w = wv[l]
                    for t in range(WAGG // LANES):
                        sl = (b0 + l, pl.ds(t * LANES, LANES))
                        rows[sl] = rows[sl] * w

            pltpu.sync_copy(rows, acc.at[idc], add=True)

        plsc.subcore_barrier()
        for r in range(rounds):
            ci = s + r * NS
            off = pl.multiple_of(ci * CHUNK, 8)

            @pl.when(ci < n_chunks)
            def _():
                pltpu.sync_copy(acc.at[pl.ds(off, CHUNK)],
                                out_hbm.at[c, pl.ds(off, CHUNK)])

    return agg_kernel(hflat, row_flat, col, ew, zeros)


def _tc_prep(x, degp):
    N, Din = x.shape
    R = 1000

    def body(x_ref, d_ref, xp_ref, dv_ref):
        deg = d_ref[0, :, 0:1] + d_ref[1, :, 0:1] + 1.0
        dinv = lax.rsqrt(deg)
        xp_ref[...] = x_ref[...] * dinv
        dv_ref[...] = dinv

    return pl.pallas_call(
        body,
        grid=(N // R,),
        in_specs=[pl.BlockSpec((R, Din), lambda i: (i, 0)),
                  pl.BlockSpec((NC, R, DEGW), lambda i: (0, i, 0))],
        out_specs=[pl.BlockSpec((R, Din), lambda i: (i, 0)),
                   pl.BlockSpec((R, 1), lambda i: (i, 0))],
        out_shape=[jax.ShapeDtypeStruct((N, Din), jnp.float32),
                   jax.ShapeDtypeStruct((N, 1), jnp.float32)],
    )(x, degp)


def _tc_layer1(acc1, xp, dinv, W1, b1):
    N, Din = xp.shape
    H = W1.shape[1]
    R = 1000

    def body(a_ref, xp_ref, dv_ref, w_ref, b_ref, out_ref):
        dv = dv_ref[...]
        z = (a_ref[0] + a_ref[1] + xp_ref[...]) * dv
        h = jnp.dot(z, w_ref[...], preferred_element_type=jnp.float32)
        h = jnp.maximum(h + b_ref[...], 0.0) * dv
        out_ref[0, :, :] = h[:, :Din]
        out_ref[1, :, :] = h[:, Din:]

    return pl.pallas_call(
        body,
        grid=(N // R,),
        in_specs=[pl.BlockSpec((2, R, Din), lambda i: (0, i, 0)),
                  pl.BlockSpec((R, Din), lambda i: (i, 0)),
                  pl.BlockSpec((R, 1), lambda i: (i, 0)),
                  pl.BlockSpec((Din, H), lambda i: (0, 0)),
                  pl.BlockSpec((1, H), lambda i: (0, 0))],
        out_specs=pl.BlockSpec((2, R, Din), lambda i: (0, i, 0)),
        out_shape=jax.ShapeDtypeStruct((2, N, Din), jnp.float32),
    )(acc1, xp, dinv, W1, b1.reshape(1, H))


def _tc_layer23(acc2, h1p, dinv, W2, b2, W3):
    _, N, Dh = h1p.shape
    H = W2.shape[0]
    Dout = W3.shape[1]
    R = 1000

    def body(a_ref, hp_ref, dv_ref, w2_ref, b2_ref, w3_ref, out_ref):
        dv = dv_ref[...]
        zl = (a_ref[0] + hp_ref[0]) * dv
        zr = (a_ref[1] + hp_ref[1]) * dv
        z = jnp.concatenate([zl, zr], axis=1)
        h2 = jnp.dot(z, w2_ref[...], preferred_element_type=jnp.float32)
        h2 = jnp.maximum(h2 + b2_ref[...], 0.0)
        m3 = jnp.dot(h2, w3_ref[...], preferred_element_type=jnp.float32)
        out_ref[...] = m3 * dv

    return pl.pallas_call(
        body,
        grid=(N // R,),
        in_specs=[pl.BlockSpec((2, R, Dh), lambda i: (0, i, 0)),
                  pl.BlockSpec((2, R, Dh), lambda i: (0, i, 0)),
                  pl.BlockSpec((R, 1), lambda i: (i, 0)),
                  pl.BlockSpec((H, H), lambda i: (0, 0)),
                  pl.BlockSpec((1, H), lambda i: (0, 0)),
                  pl.BlockSpec((H, Dout), lambda i: (0, 0))],
        out_specs=pl.BlockSpec((R, Dout), lambda i: (i, 0)),
        out_shape=jax.ShapeDtypeStruct((N, Dout), jnp.float32),
    )(acc2, h1p, dinv, W2, b2.reshape(1, H), W3)


def _tc_final(acc3, m3p, dinv, b3):
    N, Dout = m3p.shape
    R = 1000

    def body(a_ref, m_ref, dv_ref, b_ref, o_ref):
        z = (a_ref[0] + a_ref[1] + m_ref[...]) * dv_ref[...]
        o_ref[...] = jax.nn.sigmoid(z + b_ref[...])

    return pl.pallas_call(
        body,
        grid=(N // R,),
        in_specs=[pl.BlockSpec((2, R, Dout), lambda i: (0, i, 0)),
                  pl.BlockSpec((R, Dout), lambda i: (i, 0)),
                  pl.BlockSpec((R, 1), lambda i: (i, 0)),
                  pl.BlockSpec((1, Dout), lambda i: (0, 0))],
        out_specs=pl.BlockSpec((R, Dout), lambda i: (i, 0)),
        out_shape=jax.ShapeDtypeStruct((N, Dout), jnp.float32),
    )(acc3, m3p, dinv, b3.reshape(1, Dout))


def kernel(x, edge_index, edge_weight, W1, b1, W2, b2, W3, b3):
    x = x.astype(jnp.float32)
    N = x.shape[0]
    row = edge_index[0].astype(jnp.int32)
    col = edge_index[1].astype(jnp.int32)
    ew = edge_weight.astype(jnp.float32)
    E = ew.shape[0]

    pad_unit = NC * NS * EDGE_B * 2
    Ep = ((E + pad_unit - 1) // pad_unit) * pad_unit
    if Ep != E:
        row = jnp.pad(row, (0, Ep - E))
        col = jnp.pad(col, (0, Ep - E))
        ew = jnp.pad(ew, (0, Ep - E))

    zeros = jnp.zeros((N, WAGG), jnp.float32)
    degp = _sc_degree(col, ew, zeros)
    xp, dinv = _tc_prep(x, degp)

    acc1 = _sc_aggregate(xp, row, col, ew, zeros, feat_split=False)
    h1p = _tc_layer1(acc1, xp, dinv, W1, b1)

    row2 = jnp.concatenate([row, row + N])
    acc2 = _sc_aggregate(h1p.reshape(2 * N, WAGG), row2, col, ew, zeros,
                         feat_split=True)
    m3p = _tc_layer23(acc2, h1p, dinv, W2, b2, W3)

    acc3 = _sc_aggregate(m3p, row, col, ew, zeros, feat_split=False)
    return _tc_final(acc3, m3p, dinv, b3)


# speedup vs baseline: 1.6798x; 1.6798x over previous
import jax
import jax.numpy as jnp
from jax import lax
from jax.experimental import pallas as pl
from jax.experimental.pallas import tpu as pltpu
from jax.experimental.pallas import tpu_sc as plsc

NC = 2
NS = 16
LANES = 16
WAGG = 128
EDGE_B = 200
EDGE_B16 = (EDGE_B // LANES) * LANES
EDGE_TAIL = EDGE_B - EDGE_B16


def _sc_mesh():
    return plsc.VectorSubcoreMesh(
        core_axis_name="c", subcore_axis_name="s",
        num_cores=NC, num_subcores=NS)


DEGW = 128


def _sc_degree(col, ew, zeros16):
    E = col.shape[0]
    N = zeros16.shape[0]
    per = E // (NC * NS)
    CHUNK = 200
    n_chunks = N // CHUNK
    rounds = (n_chunks + NS - 1) // NS

    @pl.kernel(out_type=jax.ShapeDtypeStruct((NC, N, DEGW), jnp.float32),
               mesh=_sc_mesh(),
               scratch_types=[pltpu.VMEM_SHARED((N, DEGW), jnp.float32),
                              pltpu.VMEM((EDGE_B,), jnp.int32),
                              pltpu.VMEM((EDGE_B,), jnp.float32),
                              pltpu.VMEM((EDGE_B, DEGW), jnp.float32)])
    def deg_kernel(col_hbm, ew_hbm, z_hbm, out_hbm, acc, idc, ewv, rows):
        c = lax.axis_index("c")
        s = lax.axis_index("s")

        for r in range(rounds):
            ci = s + r * NS
            off = pl.multiple_of(ci * CHUNK, 8)

            @pl.when(ci < n_chunks)
            def _():
                pltpu.sync_copy(z_hbm.at[pl.ds(off, CHUNK)],
                                acc.at[pl.ds(off, CHUNK)])

        @pl.loop(0, EDGE_B)
        def _(b):
            for t in range(DEGW // LANES):
                rows[b, pl.ds(t * LANES, LANES)] = jnp.zeros((LANES,),
                                                             jnp.float32)

        plsc.subcore_barrier()

        base0 = (c * NS + s) * per

        @pl.loop(0, per, step=EDGE_B)
        def _(k):
            pltpu.sync_copy(col_hbm.at[pl.ds(base0 + k, EDGE_B)], idc)
            pltpu.sync_copy(ew_hbm.at[pl.ds(base0 + k, EDGE_B)], ewv)

            @pl.loop(0, EDGE_B16, step=LANES)
            def _(b0):
                wv = ewv[pl.ds(b0, LANES)]
                for l in range(LANES):
                    rows[b0 + l, pl.ds(0, LANES)] = jnp.full(
                        (LANES,), wv[l], jnp.float32)

            if EDGE_TAIL:
                wv = ewv[pl.ds(EDGE_B - LANES, LANES)]
                for l in range(LANES - EDGE_TAIL, LANES):
                    rows[EDGE_B - LANES + l, pl.ds(0, LANES)] = jnp.full(
                        (LANES,), wv[l], jnp.float32)

            pltpu.sync_copy(rows, acc.at[idc], add=True)

        plsc.subcore_barrier()
        for r in range(rounds):
            ci = s + r * NS
            off = pl.multiple_of(ci * CHUNK, 8)

            @pl.when(ci < n_chunks)
            def _():
                pltpu.sync_copy(acc.at[pl.ds(off, CHUNK)],
                                out_hbm.at[c, pl.ds(off, CHUNK)])

    return deg_kernel(col, ew, zeros16)


def _sc_aggregate(hflat, row_flat, col, ew, zeros, feat_split):
    E = col.shape[0]
    N = zeros.shape[0]
    per = E // NS if feat_split else E // (NC * NS)
    nb = per // EDGE_B
    CHUNK = 200
    n_chunks = N // CHUNK
    rounds = (n_chunks + NS - 1) // NS

    @pl.kernel(out_type=jax.ShapeDtypeStruct((NC, N, WAGG), jnp.float32),
               mesh=_sc_mesh(),
               scratch_types=[pltpu.VMEM_SHARED((N, WAGG), jnp.float32),
                              pltpu.VMEM((EDGE_B, WAGG), jnp.float32),
                              pltpu.VMEM((EDGE_B,), jnp.int32),
                              pltpu.VMEM((EDGE_B,), jnp.int32),
                              pltpu.VMEM((EDGE_B,), jnp.float32)])
    def agg_kernel(h_hbm, row_hbm, col_hbm, ew_hbm, z_hbm, out_hbm,
                   acc, rows, idr, idc, ewv):
        c = lax.axis_index("c")
        s = lax.axis_index("s")

        if feat_split:
            base0 = s * per
            roff = c * E + base0
        else:
            base0 = (c * NS + s) * per
            roff = base0

        for r in range(rounds):
            ci = s + r * NS
            off = pl.multiple_of(ci * CHUNK, 8)

            @pl.when(ci < n_chunks)
            def _():
                pltpu.sync_copy(z_hbm.at[pl.ds(off, CHUNK)],
                                acc.at[pl.ds(off, CHUNK)])
        plsc.subcore_barrier()

        @pl.loop(0, per, step=EDGE_B)
        def _(k):
            pltpu.sync_copy(row_hbm.at[pl.ds(roff + k, EDGE_B)], idr)
            pltpu.sync_copy(col_hbm.at[pl.ds(base0 + k, EDGE_B)], idc)
            pltpu.sync_copy(ew_hbm.at[pl.ds(base0 + k, EDGE_B)], ewv)
            pltpu.sync_copy(h_hbm.at[idr], rows)

            @pl.loop(0, EDGE_B16, step=LANES)
            def _(b0):
                wv = ewv[pl.ds(b0, LANES)]
                for l in range(LANES):
                    w = wv[l]
                    for t in range(WAGG // LANES):
                        sl = (b0 + l, pl.ds(t * LANES, LANES))
                        rows[sl] = rows[sl] * w

            if EDGE_TAIL:
                wv = ewv[pl.ds(EDGE_B - LANES, LANES)]
                for l in range(LANES - EDGE_TAIL, LANES):
                    w = wv[l]
                    for t in range(WAGG // LANES):
                        sl = (EDGE_B - LANES + l, pl.ds(t * LANES, LANES))
                        rows[sl] = rows[sl] * w

            pltpu.sync_copy(rows, acc.at[idc], add=True)

        plsc.subcore_barrier()
        for r in range(rounds):
            ci = s + r * NS
            off = pl.multiple_of(ci * CHUNK, 8)

            @pl.when(ci < n_chunks)
            def _():
                pltpu.sync_copy(acc.at[pl.ds(off, CHUNK)],
                                out_hbm.at[c, pl.ds(off, CHUNK)])

    return agg_kernel(hflat, row_flat, col, ew, zeros)


def _tc_prep(x, degp):
    N, Din = x.shape
    R = 1000

    def body(x_ref, d_ref, xp_ref, dv_ref):
        deg = d_ref[0, :, 0:1] + d_ref[1, :, 0:1] + 1.0
        dinv = lax.rsqrt(deg)
        xp_ref[...] = x_ref[...] * dinv
        dv_ref[...] = dinv

    return pl.pallas_call(
        body,
        grid=(N // R,),
        in_specs=[pl.BlockSpec((R, Din), lambda i: (i, 0)),
                  pl.BlockSpec((NC, R, DEGW), lambda i: (0, i, 0))],
        out_specs=[pl.BlockSpec((R, Din), lambda i: (i, 0)),
                   pl.BlockSpec((R, 1), lambda i: (i, 0))],
        out_shape=[jax.ShapeDtypeStruct((N, Din), jnp.float32),
                   jax.ShapeDtypeStruct((N, 1), jnp.float32)],
    )(x, degp)


def _tc_layer1(acc1, xp, dinv, W1, b1):
    N, Din = xp.shape
    H = W1.shape[1]
    R = 1000

    def body(a_ref, xp_ref, dv_ref, w_ref, b_ref, out_ref):
        dv = dv_ref[...]
        z = (a_ref[0] + a_ref[1] + xp_ref[...]) * dv
        h = jnp.dot(z, w_ref[...], preferred_element_type=jnp.float32)
        h = jnp.maximum(h + b_ref[...], 0.0) * dv
        out_ref[0, :, :] = h[:, :Din]
        out_ref[1, :, :] = h[:, Din:]

    return pl.pallas_call(
        body,
        grid=(N // R,),
        in_specs=[pl.BlockSpec((2, R, Din), lambda i: (0, i, 0)),
                  pl.BlockSpec((R, Din), lambda i: (i, 0)),
                  pl.BlockSpec((R, 1), lambda i: (i, 0)),
                  pl.BlockSpec((Din, H), lambda i: (0, 0)),
                  pl.BlockSpec((1, H), lambda i: (0, 0))],
        out_specs=pl.BlockSpec((2, R, Din), lambda i: (0, i, 0)),
        out_shape=jax.ShapeDtypeStruct((2, N, Din), jnp.float32),
    )(acc1, xp, dinv, W1, b1.reshape(1, H))


def _tc_layer23(acc2, h1p, dinv, W2, b2, W3):
    _, N, Dh = h1p.shape
    H = W2.shape[0]
    Dout = W3.shape[1]
    R = 1000

    def body(a_ref, hp_ref, dv_ref, w2_ref, b2_ref, w3_ref, out_ref):
        dv = dv_ref[...]
        zl = (a_ref[0] + hp_ref[0]) * dv
        zr = (a_ref[1] + hp_ref[1]) * dv
        z = jnp.concatenate([zl, zr], axis=1)
        h2 = jnp.dot(z, w2_ref[...], preferred_element_type=jnp.float32)
        h2 = jnp.maximum(h2 + b2_ref[...], 0.0)
        m3 = jnp.dot(h2, w3_ref[...], preferred_element_type=jnp.float32)
        out_ref[...] = m3 * dv

    return pl.pallas_call(
        body,
        grid=(N // R,),
        in_specs=[pl.BlockSpec((2, R, Dh), lambda i: (0, i, 0)),
                  pl.BlockSpec((2, R, Dh), lambda i: (0, i, 0)),
                  pl.BlockSpec((R, 1), lambda i: (i, 0)),
                  pl.BlockSpec((H, H), lambda i: (0, 0)),
                  pl.BlockSpec((1, H), lambda i: (0, 0)),
                  pl.BlockSpec((H, Dout), lambda i: (0, 0))],
        out_specs=pl.BlockSpec((R, Dout), lambda i: (i, 0)),
        out_shape=jax.ShapeDtypeStruct((N, Dout), jnp.float32),
    )(acc2, h1p, dinv, W2, b2.reshape(1, H), W3)


def _tc_final(acc3, m3p, dinv, b3):
    N, Dout = m3p.shape
    R = 1000

    def body(a_ref, m_ref, dv_ref, b_ref, o_ref):
        z = (a_ref[0] + a_ref[1] + m_ref[...]) * dv_ref[...]
        o_ref[...] = jax.nn.sigmoid(z + b_ref[...])

    return pl.pallas_call(
        body,
        grid=(N // R,),
        in_specs=[pl.BlockSpec((2, R, Dout), lambda i: (0, i, 0)),
                  pl.BlockSpec((R, Dout), lambda i: (i, 0)),
                  pl.BlockSpec((R, 1), lambda i: (i, 0)),
                  pl.BlockSpec((1, Dout), lambda i: (0, 0))],
        out_specs=pl.BlockSpec((R, Dout), lambda i: (i, 0)),
        out_shape=jax.ShapeDtypeStruct((N, Dout), jnp.float32),
    )(acc3, m3p, dinv, b3.reshape(1, Dout))


def kernel(x, edge_index, edge_weight, W1, b1, W2, b2, W3, b3):
    x = x.astype(jnp.float32)
    N = x.shape[0]
    row = edge_index[0].astype(jnp.int32)
    col = edge_index[1].astype(jnp.int32)
    ew = edge_weight.astype(jnp.float32)
    E = ew.shape[0]

    pad_unit = NC * NS * EDGE_B
    Ep = ((E + pad_unit - 1) // pad_unit) * pad_unit
    if Ep != E:
        row = jnp.pad(row, (0, Ep - E))
        col = jnp.pad(col, (0, Ep - E))
        ew = jnp.pad(ew, (0, Ep - E))

    zeros = jnp.zeros((N, WAGG), jnp.float32)
    degp = _sc_degree(col, ew, zeros)
    xp, dinv = _tc_prep(x, degp)

    acc1 = _sc_aggregate(xp, row, col, ew, zeros, feat_split=False)
    h1p = _tc_layer1(acc1, xp, dinv, W1, b1)

    row2 = jnp.concatenate([row, row + N])
    acc2 = _sc_aggregate(h1p.reshape(2 * N, WAGG), row2, col, ew, zeros,
                         feat_split=True)
    m3p = _tc_layer23(acc2, h1p, dinv, W2, b2, W3)

    acc3 = _sc_aggregate(m3p, row, col, ew, zeros, feat_split=False)
    return _tc_final(acc3, m3p, dinv, b3)
